# Initial kernel scaffold; baseline (speedup 1.0000x reference)
import jax
import jax.numpy as jnp
from jax.experimental import pallas as pl


def kernel(embeddings, labels, embedding_memory, label_memory):
    raise NotImplementedError("write your pallas kernel here")


# speedup vs baseline: 1.3126x; 1.3126x over previous
import functools

import jax
import jax.numpy as jnp
from jax.experimental import pallas as pl
from jax.experimental.pallas import tpu as pltpu

_B = 1024
_D = 128
_M = 16384
_BLK = 1024
_NBLK = _M // _BLK


def _loss_block(x_ref, lcol_ref, lrow_ref, mem_ref, lmem_ref, out_ref, acc_ref):
    j = pl.program_id(0)

    @pl.when(j == 0)
    def _init():
        acc_ref[0] = 0.0
        acc_ref[1] = 0.0
        acc_ref[2] = 0.0
        acc_ref[3] = 0.0

    x = x_ref[...]
    is_batch = j == 0
    y = jnp.where(is_batch, x, mem_ref[...])
    ly = jnp.where(is_batch, lrow_ref[...], lmem_ref[...])

    xsq = jnp.sum(x * x, axis=1, keepdims=True)
    ysq = jax.lax.dot_general(
        jnp.ones((1, _D), jnp.float32), y * y,
        (((1,), (1,)), ((), ())),
        precision=jax.lax.Precision.HIGHEST,
        preferred_element_type=jnp.float32)
    g = jax.lax.dot_general(
        x, y, (((1,), (1,)), ((), ())),
        precision=jax.lax.Precision.HIGHEST,
        preferred_element_type=jnp.float32)
    sq = (xsq + ysq) - 2.0 * g
    d = jnp.sqrt(jnp.maximum(sq, 1e-12))

    matches = lcol_ref[...] == ly
    rows = jax.lax.broadcasted_iota(jnp.int32, (_B, _BLK), 0)
    cols = jax.lax.broadcasted_iota(jnp.int32, (_B, _BLK), 1)
    selfm = jnp.logical_and(is_batch, rows == cols)
    pos_mask = jnp.logical_and(matches, jnp.logical_not(selfm))
    neg_mask = jnp.logical_not(matches)

    neg_elt = jnp.maximum(1.0 - d, 0.0)
    one = jnp.float32(1.0)
    zero = jnp.float32(0.0)
    pos_sum = jnp.sum(jnp.where(pos_mask, d, zero))
    pos_cnt = jnp.sum(jnp.where(jnp.logical_and(pos_mask, d > 0.0), one, zero))
    neg_sum = jnp.sum(jnp.where(neg_mask, neg_elt, zero))
    neg_cnt = jnp.sum(
        jnp.where(jnp.logical_and(neg_mask, neg_elt > 0.0), one, zero))

    acc_ref[0] = acc_ref[0] + pos_sum
    acc_ref[1] = acc_ref[1] + pos_cnt
    acc_ref[2] = acc_ref[2] + neg_sum
    acc_ref[3] = acc_ref[3] + neg_cnt

    @pl.when(j == _NBLK - 1)
    def _finish():
        pos_loss = acc_ref[0] / jnp.maximum(acc_ref[1], 1.0)
        neg_loss = acc_ref[2] / jnp.maximum(acc_ref[3], 1.0)
        out_ref[0] = pos_loss + neg_loss


@functools.partial(jax.jit, static_argnames=())
def kernel(embeddings, labels, embedding_memory, label_memory):
    lcol = labels.reshape(_B, 1)
    lrow = labels.reshape(1, _B)
    lmem = label_memory.reshape(1, _M)
    out = pl.pallas_call(
        _loss_block,
        grid=(_NBLK,),
        in_specs=[
            pl.BlockSpec((_B, _D), lambda j: (0, 0)),
            pl.BlockSpec((_B, 1), lambda j: (0, 0)),
            pl.BlockSpec((1, _B), lambda j: (0, 0)),
            pl.BlockSpec((_BLK, _D), lambda j: (j, 0)),
            pl.BlockSpec((1, _BLK), lambda j: (0, j)),
        ],
        out_specs=pl.BlockSpec(memory_space=pltpu.SMEM),
        out_shape=jax.ShapeDtypeStruct((1,), jnp.float32),
        scratch_shapes=[pltpu.SMEM((4,), jnp.float32)],
    )(embeddings, lcol, lrow, embedding_memory, lmem)
    return out[0]


# speedup vs baseline: 2.1592x; 1.6449x over previous
import functools

import jax
import jax.numpy as jnp
from jax.experimental import pallas as pl
from jax.experimental.pallas import tpu as pltpu

_B = 1024
_D = 128
_M = 16384
_BLK = 1024
_NBLK = _M // _BLK


def _loss_block(x_ref, lcol_ref, lrow_ref, mem_ref, lmem_ref, out_ref, acc_ref):
    j = pl.program_id(0)

    @pl.when(j == 0)
    def _init():
        acc_ref[0] = 0.0
        acc_ref[1] = 0.0
        acc_ref[2] = 0.0
        acc_ref[3] = 0.0

    x = x_ref[...]
    is_batch = j == 0
    y = jnp.where(is_batch, x, mem_ref[...])
    ly = jnp.where(is_batch, lrow_ref[...], lmem_ref[...])

    xsq = jnp.sum(x * x, axis=1, keepdims=True)
    ysq = jax.lax.dot_general(
        jnp.ones((1, _D), jnp.float32), y * y,
        (((1,), (1,)), ((), ())),
        preferred_element_type=jnp.float32)
    g = jax.lax.dot_general(
        x, y, (((1,), (1,)), ((), ())),
        preferred_element_type=jnp.float32)
    d = jnp.sqrt(jnp.maximum((xsq + ysq) - 2.0 * g, 1e-12))

    matches = lcol_ref[...] == ly
    one = jnp.float32(1.0)
    zero = jnp.float32(0.0)
    pos_sum = jnp.sum(jnp.where(matches, d, zero))
    pos_cnt = jnp.sum(jnp.where(matches, one, zero))
    neg_elt = jnp.maximum(1.0 - d, 0.0)
    neg_sel = jnp.where(matches, zero, neg_elt)
    neg_sum = jnp.sum(neg_sel)
    neg_cnt = jnp.sum(jnp.where(neg_sel > zero, one, zero))

    acc_ref[0] = acc_ref[0] + pos_sum
    acc_ref[1] = acc_ref[1] + pos_cnt
    acc_ref[2] = acc_ref[2] + neg_sum
    acc_ref[3] = acc_ref[3] + neg_cnt

    @pl.when(is_batch)
    def _self_correction():
        rows = jax.lax.broadcasted_iota(jnp.int32, (_B, _BLK), 0)
        cols = jax.lax.broadcasted_iota(jnp.int32, (_B, _BLK), 1)
        diag_sum = jnp.sum(jnp.where(rows == cols, d, zero))
        acc_ref[0] = acc_ref[0] - diag_sum
        acc_ref[1] = acc_ref[1] - jnp.float32(_B)

    @pl.when(j == _NBLK - 1)
    def _finish():
        pos_loss = acc_ref[0] / jnp.maximum(acc_ref[1], 1.0)
        neg_loss = acc_ref[2] / jnp.maximum(acc_ref[3], 1.0)
        out_ref[0] = pos_loss + neg_loss


@functools.partial(jax.jit, static_argnames=())
def kernel(embeddings, labels, embedding_memory, label_memory):
    lcol = labels.reshape(_B, 1)
    lrow = labels.reshape(1, _B)
    lmem = label_memory.reshape(1, _M)
    out = pl.pallas_call(
        _loss_block,
        grid=(_NBLK,),
        in_specs=[
            pl.BlockSpec((_B, _D), lambda j: (0, 0)),
            pl.BlockSpec((_B, 1), lambda j: (0, 0)),
            pl.BlockSpec((1, _B), lambda j: (0, 0)),
            pl.BlockSpec((_BLK, _D), lambda j: (j, 0)),
            pl.BlockSpec((1, _BLK), lambda j: (0, j)),
        ],
        out_specs=pl.BlockSpec(memory_space=pltpu.SMEM),
        out_shape=jax.ShapeDtypeStruct((1,), jnp.float32),
        scratch_shapes=[pltpu.SMEM((4,), jnp.float32)],
    )(embeddings, lcol, lrow, embedding_memory, lmem)
    return out[0]


# speedup vs baseline: 2.6544x; 1.2294x over previous
import functools

import jax
import jax.numpy as jnp
from jax.experimental import pallas as pl
from jax.experimental.pallas import tpu as pltpu

_B = 1024
_D = 128
_M = 16384
_BLK = 1024
_NBLK = _M // _BLK


def _loss_block(x_ref, lcol_ref, lrow_ref, mem_ref, lmem_ref, out_ref, acc_ref):
    j = pl.program_id(0)

    @pl.when(j == 0)
    def _init():
        acc_ref[0] = 0.0
        acc_ref[1] = 0.0
        acc_ref[2] = 0.0
        acc_ref[3] = 0.0

    x = x_ref[...]
    is_batch = j == 0
    y = jnp.where(is_batch, x, mem_ref[...])
    ly = jnp.where(is_batch, lrow_ref[...], lmem_ref[...])

    xsq = jnp.sum(x * x, axis=1, keepdims=True)
    ysq = jnp.sum(y * y, axis=1, keepdims=True)
    xa = jnp.concatenate(
        [x * jnp.float32(-2.0), xsq, jnp.ones((_B, 1), jnp.float32)], axis=1)
    ya = jnp.concatenate(
        [y, jnp.ones((_BLK, 1), jnp.float32), ysq], axis=1)
    d2 = jax.lax.dot_general(
        xa, ya, (((1,), (1,)), ((), ())),
        preferred_element_type=jnp.float32)
    m = jnp.maximum(d2, 1e-12)
    d = m * jax.lax.rsqrt(m)

    matches = lcol_ref[...] == ly
    one = jnp.float32(1.0)
    zero = jnp.float32(0.0)
    neg_live = jnp.logical_and(jnp.logical_not(matches), d2 < one)

    pos_sum = jnp.sum(jnp.where(matches, d, zero))
    pos_cnt = jnp.sum(jnp.where(matches, one, zero))
    negd_sum = jnp.sum(jnp.where(neg_live, d, zero))
    neg_cnt = jnp.sum(jnp.where(neg_live, one, zero))

    acc_ref[0] = acc_ref[0] + pos_sum
    acc_ref[1] = acc_ref[1] + pos_cnt
    acc_ref[2] = acc_ref[2] + negd_sum
    acc_ref[3] = acc_ref[3] + neg_cnt

    @pl.when(is_batch)
    def _self_correction():
        rows = jax.lax.broadcasted_iota(jnp.int32, (_B, _BLK), 0)
        cols = jax.lax.broadcasted_iota(jnp.int32, (_B, _BLK), 1)
        diag_sum = jnp.sum(jnp.where(rows == cols, d, zero))
        acc_ref[0] = acc_ref[0] - diag_sum
        acc_ref[1] = acc_ref[1] - jnp.float32(_B)

    @pl.when(j == _NBLK - 1)
    def _finish():
        pos_loss = acc_ref[0] / jnp.maximum(acc_ref[1], 1.0)
        neg_loss = (acc_ref[3] - acc_ref[2]) / jnp.maximum(acc_ref[3], 1.0)
        out_ref[0] = pos_loss + neg_loss


@functools.partial(jax.jit, static_argnames=())
def kernel(embeddings, labels, embedding_memory, label_memory):
    lcol = labels.reshape(_B, 1)
    lrow = labels.reshape(1, _B)
    lmem = label_memory.reshape(1, _M)
    out = pl.pallas_call(
        _loss_block,
        grid=(_NBLK,),
        in_specs=[
            pl.BlockSpec((_B, _D), lambda j: (0, 0)),
            pl.BlockSpec((_B, 1), lambda j: (0, 0)),
            pl.BlockSpec((1, _B), lambda j: (0, 0)),
            pl.BlockSpec((_BLK, _D), lambda j: (j, 0)),
            pl.BlockSpec((1, _BLK), lambda j: (0, j)),
        ],
        out_specs=pl.BlockSpec(memory_space=pltpu.SMEM),
        out_shape=jax.ShapeDtypeStruct((1,), jnp.float32),
        scratch_shapes=[pltpu.SMEM((4,), jnp.float32)],
    )(embeddings, lcol, lrow, embedding_memory, lmem)
    return out[0]


# speedup vs baseline: 2.9776x; 1.1218x over previous
import functools

import jax
import jax.numpy as jnp
from jax.experimental import pallas as pl
from jax.experimental.pallas import tpu as pltpu

_B = 1024
_D = 128
_M = 16384
_BLK = 2048
_NBLK = _M // _BLK


def _loss_block(x_ref, lcol_ref, lrow_ref, mem_ref, lmem_ref, out_ref,
                xa_ref, acc_ref):
    j = pl.program_id(0)
    is_batch = j == 0

    @pl.when(is_batch)
    def _init():
        acc_ref[0] = 0.0
        acc_ref[1] = 0.0
        acc_ref[2] = 0.0
        acc_ref[3] = 0.0
        x = x_ref[...]
        xsq = jnp.sum(x * x, axis=1, keepdims=True)
        xa_ref[...] = jnp.concatenate(
            [x * jnp.float32(-2.0), xsq, jnp.ones((_B, 1), jnp.float32)],
            axis=1)

    y0 = jnp.concatenate(
        [x_ref[...], mem_ref[pl.ds(_B, _BLK - _B), :]], axis=0)
    ly0 = jnp.concatenate(
        [lrow_ref[...], lmem_ref[:, pl.ds(_B, _BLK - _B)]], axis=1)
    y = jnp.where(is_batch, y0, mem_ref[...])
    ly = jnp.where(is_batch, ly0, lmem_ref[...])

    ysq = jnp.sum(y * y, axis=1, keepdims=True)
    ya = jnp.concatenate(
        [y, jnp.ones((_BLK, 1), jnp.float32), ysq], axis=1)
    d2 = jax.lax.dot_general(
        xa_ref[...], ya, (((1,), (1,)), ((), ())),
        preferred_element_type=jnp.float32)
    m = jnp.maximum(d2, 1e-12)
    d = m * jax.lax.rsqrt(m)

    one = jnp.float32(1.0)
    zero = jnp.float32(0.0)
    mf = jnp.where(lcol_ref[...] == ly, one, zero)
    ltf = jnp.where(d2 < one, one, zero)
    nf = ltf * (one - mf)

    acc_ref[0] = acc_ref[0] + jnp.sum(d * mf)
    acc_ref[1] = acc_ref[1] + jnp.sum(mf)
    acc_ref[2] = acc_ref[2] + jnp.sum(d * nf)
    acc_ref[3] = acc_ref[3] + jnp.sum(nf)

    @pl.when(is_batch)
    def _self_correction():
        rows = jax.lax.broadcasted_iota(jnp.int32, (_B, _BLK), 0)
        cols = jax.lax.broadcasted_iota(jnp.int32, (_B, _BLK), 1)
        diag_sum = jnp.sum(jnp.where(rows == cols, d, zero))
        acc_ref[0] = acc_ref[0] - diag_sum
        acc_ref[1] = acc_ref[1] - jnp.float32(_B)

    @pl.when(j == _NBLK - 1)
    def _finish():
        pos_loss = acc_ref[0] / jnp.maximum(acc_ref[1], 1.0)
        neg_loss = (acc_ref[3] - acc_ref[2]) / jnp.maximum(acc_ref[3], 1.0)
        out_ref[0] = pos_loss + neg_loss


@functools.partial(jax.jit, static_argnames=())
def kernel(embeddings, labels, embedding_memory, label_memory):
    lcol = labels.reshape(_B, 1)
    lrow = labels.reshape(1, _B)
    lmem = label_memory.reshape(1, _M)
    out = pl.pallas_call(
        _loss_block,
        grid=(_NBLK,),
        in_specs=[
            pl.BlockSpec((_B, _D), lambda j: (0, 0)),
            pl.BlockSpec((_B, 1), lambda j: (0, 0)),
            pl.BlockSpec((1, _B), lambda j: (0, 0)),
            pl.BlockSpec((_BLK, _D), lambda j: (j, 0)),
            pl.BlockSpec((1, _BLK), lambda j: (0, j)),
        ],
        out_specs=pl.BlockSpec(memory_space=pltpu.SMEM),
        out_shape=jax.ShapeDtypeStruct((1,), jnp.float32),
        scratch_shapes=[
            pltpu.VMEM((_B, _D + 2), jnp.float32),
            pltpu.SMEM((4,), jnp.float32),
        ],
    )(embeddings, lcol, lrow, embedding_memory, lmem)
    return out[0]


# speedup vs baseline: 3.0861x; 1.0364x over previous
import functools

import jax
import jax.numpy as jnp
from jax.experimental import pallas as pl
from jax.experimental.pallas import tpu as pltpu

_B = 1024
_D = 128
_M = 16384
_BLK = 2048
_NBLK = _M // _BLK


def _loss_block(x_ref, lab_ref, mem_ref, lmem_ref, out_ref,
                xa_ref, lcol_ref, acc_ref):
    j = pl.program_id(0)
    is_batch = j == 0

    @pl.when(is_batch)
    def _init():
        acc_ref[0] = 0.0
        acc_ref[1] = 0.0
        acc_ref[2] = 0.0
        acc_ref[3] = 0.0
        x = x_ref[...]
        xsq = jnp.sum(x * x, axis=1, keepdims=True)
        xa_ref[...] = jnp.concatenate(
            [x * jnp.float32(-2.0), xsq, jnp.ones((_B, 1), jnp.float32)],
            axis=1)
        lcol_ref[...] = lab_ref[...].reshape(1, _B).T

    y0 = jnp.concatenate(
        [x_ref[...], mem_ref[pl.ds(_B, _BLK - _B), :]], axis=0)
    ly0 = jnp.concatenate(
        [lab_ref[...].reshape(1, _B),
         lmem_ref[pl.ds(_B, _BLK - _B)].reshape(1, _BLK - _B)], axis=1)
    y = jnp.where(is_batch, y0, mem_ref[...])
    ly = jnp.where(is_batch, ly0, lmem_ref[...].reshape(1, _BLK))

    ysq = jnp.sum(y * y, axis=1, keepdims=True)
    ya = jnp.concatenate(
        [y, jnp.ones((_BLK, 1), jnp.float32), ysq], axis=1)
    d2 = jax.lax.dot_general(
        xa_ref[...], ya, (((1,), (1,)), ((), ())),
        preferred_element_type=jnp.float32)
    m = jnp.maximum(d2, 1e-12)
    d = m * jax.lax.rsqrt(m)

    one = jnp.float32(1.0)
    zero = jnp.float32(0.0)
    mf = jnp.where(lcol_ref[...] == ly, one, zero)
    ltf = jnp.where(d2 < one, one, zero)
    nf = ltf * (one - mf)

    acc_ref[0] = acc_ref[0] + jnp.sum(d * mf)
    acc_ref[1] = acc_ref[1] + jnp.sum(mf)
    acc_ref[2] = acc_ref[2] + jnp.sum(d * nf)
    acc_ref[3] = acc_ref[3] + jnp.sum(nf)

    @pl.when(is_batch)
    def _self_correction():
        rows = jax.lax.broadcasted_iota(jnp.int32, (_B, _BLK), 0)
        cols = jax.lax.broadcasted_iota(jnp.int32, (_B, _BLK), 1)
        diag_sum = jnp.sum(jnp.where(rows == cols, d, zero))
        acc_ref[0] = acc_ref[0] - diag_sum
        acc_ref[1] = acc_ref[1] - jnp.float32(_B)

    @pl.when(j == _NBLK - 1)
    def _finish():
        pos_loss = acc_ref[0] / jnp.maximum(acc_ref[1], 1.0)
        neg_loss = (acc_ref[3] - acc_ref[2]) / jnp.maximum(acc_ref[3], 1.0)
        out_ref[0] = pos_loss + neg_loss


@functools.partial(jax.jit, static_argnames=())
def kernel(embeddings, labels, embedding_memory, label_memory):
    out = pl.pallas_call(
        _loss_block,
        grid=(_NBLK,),
        in_specs=[
            pl.BlockSpec((_B, _D), lambda j: (0, 0)),
            pl.BlockSpec((_B,), lambda j: (0,)),
            pl.BlockSpec((_BLK, _D), lambda j: (j, 0)),
            pl.BlockSpec((_BLK,), lambda j: (j,)),
        ],
        out_specs=pl.BlockSpec(memory_space=pltpu.SMEM),
        out_shape=jax.ShapeDtypeStruct((1,), jnp.float32),
        scratch_shapes=[
            pltpu.VMEM((_B, _D + 2), jnp.float32),
            pltpu.VMEM((_B, 1), jnp.int32),
            pltpu.SMEM((4,), jnp.float32),
        ],
    )(embeddings, labels, embedding_memory, label_memory)
    return out[0]
